# Initial kernel scaffold; baseline (speedup 1.0000x reference)
import jax
import jax.numpy as jnp
from jax.experimental import pallas as pl


def kernel(x, Wg, bg, We, be):
    raise NotImplementedError("write your pallas kernel here")


# speedup vs baseline: 6.0557x; 6.0557x over previous
import functools

import jax
import jax.numpy as jnp
from jax.experimental import pallas as pl
from jax.experimental.pallas import tpu as pltpu

B, S, H, E, TOP_K = 2, 2048, 1024, 8, 2
T = B * S
BM = 512


def _moe_body(x_ref, wg_ref, bg_ref, wet_ref, be_ref, out_ref):
    xb = x_ref[...]
    logits = jax.lax.dot_general(
        xb, wg_ref[...], (((1,), (1,)), ((), ())),
        preferred_element_type=jnp.float32)
    logits = logits + bg_ref[...]
    m = jnp.max(logits, axis=1, keepdims=True)
    ex = jnp.exp(logits - m)
    gates = ex / jnp.sum(ex, axis=1, keepdims=True)
    g1 = jnp.max(gates, axis=1, keepdims=True)
    gates_no1 = jnp.where(gates == g1, -1.0, gates)
    g2 = jnp.max(gates_no1, axis=1, keepdims=True)
    w = jnp.where(gates >= g2, gates, 0.0)
    acc = jnp.zeros((BM, H), dtype=jnp.float32)
    for e in range(E):
        mm = jnp.dot(xb, wet_ref[e], preferred_element_type=jnp.float32)
        acc = acc + w[:, e:e + 1] * (mm + be_ref[e][None, :])
    out_ref[...] = acc


@jax.jit
def _moe(xf, wg, bg2, wet, be):
    return pl.pallas_call(
        _moe_body,
        grid=(T // BM,),
        in_specs=[
            pl.BlockSpec((BM, H), lambda i: (i, 0)),
            pl.BlockSpec((E, H), lambda i: (0, 0)),
            pl.BlockSpec((1, E), lambda i: (0, 0)),
            pl.BlockSpec((E, H, H), lambda i: (0, 0, 0)),
            pl.BlockSpec((E, H), lambda i: (0, 0)),
        ],
        out_specs=pl.BlockSpec((BM, H), lambda i: (i, 0)),
        out_shape=jax.ShapeDtypeStruct((T, H), jnp.float32),
        compiler_params=pltpu.CompilerParams(
            dimension_semantics=("parallel",),
        ),
    )(xf, wg, bg2, wet, be)


def kernel(x, Wg, bg, We, be):
    xf = x.reshape(T, H)
    wet = We.transpose(0, 2, 1)
    out = _moe(xf, Wg, bg.reshape(1, E), wet, be)
    return out.reshape(B, S, H)


# speedup vs baseline: 6.1146x; 1.0097x over previous
import functools

import jax
import jax.numpy as jnp
from jax.experimental import pallas as pl
from jax.experimental.pallas import tpu as pltpu

B, S, H, E, TOP_K = 2, 2048, 1024, 8, 2
T = B * S
BM = 512


def _moe_body(x_ref, wg_ref, bg_ref, wet_ref, be_ref, out_ref):
    xb = x_ref[...]
    logits = jax.lax.dot_general(
        xb, wg_ref[...], (((1,), (1,)), ((), ())),
        preferred_element_type=jnp.float32)
    logits = logits + bg_ref[...]
    m = jnp.max(logits, axis=1, keepdims=True)
    ex = jnp.exp(logits - m)
    gates = ex / jnp.sum(ex, axis=1, keepdims=True)
    g1 = jnp.max(gates, axis=1, keepdims=True)
    gates_no1 = jnp.where(gates == g1, -1.0, gates)
    g2 = jnp.max(gates_no1, axis=1, keepdims=True)
    w = jnp.where(gates >= g2, gates, 0.0)
    acc = jnp.zeros((BM, H), dtype=jnp.float32)
    xb16 = xb.astype(jnp.bfloat16)
    for e in range(E):
        mm = jnp.dot(xb16, wet_ref[e].astype(jnp.bfloat16),
                     preferred_element_type=jnp.float32)
        acc = acc + w[:, e:e + 1] * (mm + be_ref[e][None, :])
    out_ref[...] = acc


@jax.jit
def _moe(xf, wg, bg2, wet, be):
    return pl.pallas_call(
        _moe_body,
        grid=(T // BM,),
        in_specs=[
            pl.BlockSpec((BM, H), lambda i: (i, 0)),
            pl.BlockSpec((E, H), lambda i: (0, 0)),
            pl.BlockSpec((1, E), lambda i: (0, 0)),
            pl.BlockSpec((E, H, H), lambda i: (0, 0, 0)),
            pl.BlockSpec((E, H), lambda i: (0, 0)),
        ],
        out_specs=pl.BlockSpec((BM, H), lambda i: (i, 0)),
        out_shape=jax.ShapeDtypeStruct((T, H), jnp.float32),
        compiler_params=pltpu.CompilerParams(
            dimension_semantics=("parallel",),
        ),
    )(xf, wg, bg2, wet, be)


def kernel(x, Wg, bg, We, be):
    xf = x.reshape(T, H)
    wet = We.transpose(0, 2, 1)
    out = _moe(xf, Wg, bg.reshape(1, E), wet, be)
    return out.reshape(B, S, H)


# speedup vs baseline: 7.2006x; 1.1776x over previous
import functools

import jax
import jax.numpy as jnp
from jax.experimental import pallas as pl
from jax.experimental.pallas import tpu as pltpu

B, S, H, E, TOP_K = 2, 2048, 1024, 8, 2
T = B * S
BM = 512


def _moe_body(x_ref, wg_ref, bg_ref, wet_ref, be_ref, out_ref):
    xb = x_ref[...]
    logits = jax.lax.dot_general(
        xb, wg_ref[...], (((1,), (1,)), ((), ())),
        preferred_element_type=jnp.float32)
    logits = logits + bg_ref[...]
    m = jnp.max(logits, axis=1, keepdims=True)
    ex = jnp.exp(logits - m)
    gates = ex / jnp.sum(ex, axis=1, keepdims=True)
    g1 = jnp.max(gates, axis=1, keepdims=True)
    gates_no1 = jnp.where(gates == g1, -1.0, gates)
    g2 = jnp.max(gates_no1, axis=1, keepdims=True)
    w = jnp.where(gates >= g2, gates, 0.0)
    acc = jnp.dot(w, be_ref[...], preferred_element_type=jnp.float32)
    xb16 = xb.astype(jnp.bfloat16)
    for e in range(E):
        mm = jnp.dot(xb16, wet_ref[e], preferred_element_type=jnp.float32)
        acc = acc + w[:, e:e + 1] * mm
    out_ref[...] = acc


@jax.jit
def _moe(xf, wg, bg2, wet, be):
    return pl.pallas_call(
        _moe_body,
        grid=(T // BM,),
        in_specs=[
            pl.BlockSpec((BM, H), lambda i: (i, 0)),
            pl.BlockSpec((E, H), lambda i: (0, 0)),
            pl.BlockSpec((1, E), lambda i: (0, 0)),
            pl.BlockSpec((E, H, H), lambda i: (0, 0, 0)),
            pl.BlockSpec((E, H), lambda i: (0, 0)),
        ],
        out_specs=pl.BlockSpec((BM, H), lambda i: (i, 0)),
        out_shape=jax.ShapeDtypeStruct((T, H), jnp.float32),
        compiler_params=pltpu.CompilerParams(
            dimension_semantics=("parallel",),
        ),
    )(xf, wg, bg2, wet, be)


def kernel(x, Wg, bg, We, be):
    xf = x.reshape(T, H)
    wet = We.transpose(0, 2, 1).astype(jnp.bfloat16)
    out = _moe(xf, Wg, bg.reshape(1, E), wet, be)
    return out.reshape(B, S, H)
